# baseline (device time: 226838 ns/iter reference)
import jax
import jax.numpy as jnp
from jax import lax
from jax.experimental import pallas as pl
from jax.experimental.pallas import tpu as pltpu

N_DEV = 8
N_TOK = 2048
D_IN = 512
D_OUT = 1024
E_LOCAL = 8
CHUNK = N_TOK // N_DEV
N_STEP = N_DEV - 1


def _moe_body(x_ref, w_ref, we_ref, out_ref, comm_ref, send_sems, recv_sems):
    my = lax.axis_index("i")
    left = jnp.remainder(my - 1, N_DEV)
    right = jnp.remainder(my + 1, N_DEV)

    barrier_sem = pltpu.get_barrier_semaphore()
    for nbr in (left, right):
        pl.semaphore_signal(
            barrier_sem, inc=1, device_id=(nbr,),
            device_id_type=pl.DeviceIdType.MESH,
        )

    for c in range(N_DEV):
        rows = pl.ds(c * CHUNK, CHUNK)
        xc = x_ref[rows, :].astype(jnp.float32)
        wc = w_ref[rows, :]
        acc = jnp.zeros((CHUNK, D_OUT), jnp.float32)
        for e in range(E_LOCAL):
            xw = (xc * wc[:, e][:, None]).astype(jnp.bfloat16)
            acc += jnp.dot(xw, we_ref[e], preferred_element_type=jnp.float32)
        out_ref[rows, :] = acc

    pl.semaphore_wait(barrier_sem, 2)

    for s in range(N_STEP):
        send_c = jnp.remainder(my - s, N_DEV)
        recv_c = jnp.remainder(my - s - 1, N_DEV)
        rdma = pltpu.make_async_remote_copy(
            src_ref=out_ref.at[pl.ds(send_c * CHUNK, CHUNK), :],
            dst_ref=comm_ref.at[s],
            send_sem=send_sems.at[s],
            recv_sem=recv_sems.at[s],
            device_id=(right,),
            device_id_type=pl.DeviceIdType.MESH,
        )
        rdma.start()
        rdma.wait()
        rrows = pl.ds(recv_c * CHUNK, CHUNK)
        out_ref[rrows, :] = out_ref[rrows, :] + comm_ref[s]

    for t in range(N_STEP):
        s = N_STEP + t
        send_c = jnp.remainder(my + 1 - t, N_DEV)
        recv_c = jnp.remainder(my - t, N_DEV)
        rdma = pltpu.make_async_remote_copy(
            src_ref=out_ref.at[pl.ds(send_c * CHUNK, CHUNK), :],
            dst_ref=comm_ref.at[s],
            send_sem=send_sems.at[s],
            recv_sem=recv_sems.at[s],
            device_id=(right,),
            device_id_type=pl.DeviceIdType.MESH,
        )
        rdma.start()
        rdma.wait()
        out_ref[pl.ds(recv_c * CHUNK, CHUNK), :] = comm_ref[s]


def kernel(x, router_W, route_idx, expert_W):
    my = lax.axis_index("i")

    scores = x @ router_W
    probs = jax.nn.softmax(scores, axis=-1)
    g = jnp.take_along_axis(probs, route_idx, axis=1)
    gn = g / g.sum(axis=1, keepdims=True)
    gids = my * E_LOCAL + jnp.arange(E_LOCAL, dtype=route_idx.dtype)
    w = jnp.sum(
        (route_idx[:, :, None] == gids[None, None, :]) * gn[:, :, None],
        axis=1,
    )

    xb = x.astype(jnp.bfloat16)
    web = expert_W.astype(jnp.bfloat16)

    return pl.pallas_call(
        _moe_body,
        out_shape=jax.ShapeDtypeStruct((N_TOK, D_OUT), jnp.float32),
        in_specs=[pl.BlockSpec(memory_space=pltpu.VMEM)] * 3,
        out_specs=pl.BlockSpec(memory_space=pltpu.VMEM),
        scratch_shapes=[
            pltpu.VMEM((2 * N_STEP, CHUNK, D_OUT), jnp.float32),
            pltpu.SemaphoreType.DMA((2 * N_STEP,)),
            pltpu.SemaphoreType.DMA((2 * N_STEP,)),
        ],
        compiler_params=pltpu.CompilerParams(collective_id=0),
    )(xb, w, web)


# device time: 219269 ns/iter; 1.0345x vs baseline; 1.0345x over previous
import jax
import jax.numpy as jnp
from jax import lax
from jax.experimental import pallas as pl
from jax.experimental.pallas import tpu as pltpu

N_DEV = 8
N_TOK = 2048
D_IN = 512
D_OUT = 1024
N_EXP = 64
E_LOCAL = 8
CHUNK = N_TOK // N_DEV
N_STEP = N_DEV - 1


def _moe_body(x_ref, rw_ref, idx_ref, we_ref, out_ref,
              comm_ref, send_sems, recv_sems):
    my = lax.axis_index("i")
    left = jnp.remainder(my - 1, N_DEV)
    right = jnp.remainder(my + 1, N_DEV)

    barrier_sem = pltpu.get_barrier_semaphore()
    for nbr in (left, right):
        pl.semaphore_signal(
            barrier_sem, inc=1, device_id=(nbr,),
            device_id_type=pl.DeviceIdType.MESH,
        )

    x = x_ref[:, :]
    scores = jnp.dot(x, rw_ref[:, :], preferred_element_type=jnp.float32)
    e = jnp.exp(scores - jnp.max(scores, axis=1, keepdims=True))
    idx0 = idx_ref[:, 0:1]
    idx1 = idx_ref[:, 1:2]
    iota64 = lax.broadcasted_iota(jnp.int32, (N_TOK, N_EXP), 1)
    g0 = jnp.sum(jnp.where(iota64 == idx0, e, 0.0), axis=1, keepdims=True)
    g1 = jnp.sum(jnp.where(iota64 == idx1, e, 0.0), axis=1, keepdims=True)
    gs = g0 + g1
    iota8 = lax.broadcasted_iota(jnp.int32, (N_TOK, E_LOCAL), 1) + my * E_LOCAL
    w = (jnp.where(iota8 == idx0, g0 / gs, 0.0)
         + jnp.where(iota8 == idx1, g1 / gs, 0.0))

    xb = x.astype(jnp.bfloat16)
    for c in range(N_DEV):
        rows = pl.ds(c * CHUNK, CHUNK)
        xc = x[c * CHUNK:(c + 1) * CHUNK, :]
        wc = w[c * CHUNK:(c + 1) * CHUNK, :]
        acc = jnp.zeros((CHUNK, D_OUT), jnp.float32)
        for ex in range(E_LOCAL):
            xw = (xc * wc[:, ex][:, None]).astype(jnp.bfloat16)
            acc += jnp.dot(xw, we_ref[ex], preferred_element_type=jnp.float32)
        out_ref[rows, :] = acc
    del xb

    pl.semaphore_wait(barrier_sem, 2)

    for s in range(N_STEP):
        send_c = jnp.remainder(my - s, N_DEV)
        recv_c = jnp.remainder(my - s - 1, N_DEV)
        rdma = pltpu.make_async_remote_copy(
            src_ref=out_ref.at[pl.ds(send_c * CHUNK, CHUNK), :],
            dst_ref=comm_ref.at[s],
            send_sem=send_sems.at[s],
            recv_sem=recv_sems.at[s],
            device_id=(right,),
            device_id_type=pl.DeviceIdType.MESH,
        )
        rdma.start()
        rdma.wait()
        rrows = pl.ds(recv_c * CHUNK, CHUNK)
        out_ref[rrows, :] = out_ref[rrows, :] + comm_ref[s]

    for t in range(N_STEP):
        s = N_STEP + t
        send_c = jnp.remainder(my + 1 - t, N_DEV)
        recv_c = jnp.remainder(my - t, N_DEV)
        rdma = pltpu.make_async_remote_copy(
            src_ref=out_ref.at[pl.ds(send_c * CHUNK, CHUNK), :],
            dst_ref=comm_ref.at[s],
            send_sem=send_sems.at[s],
            recv_sem=recv_sems.at[s],
            device_id=(right,),
            device_id_type=pl.DeviceIdType.MESH,
        )
        rdma.start()
        rdma.wait()
        out_ref[pl.ds(recv_c * CHUNK, CHUNK), :] = comm_ref[s]


def kernel(x, router_W, route_idx, expert_W):
    web = expert_W.astype(jnp.bfloat16)
    return pl.pallas_call(
        _moe_body,
        out_shape=jax.ShapeDtypeStruct((N_TOK, D_OUT), jnp.float32),
        in_specs=[pl.BlockSpec(memory_space=pltpu.VMEM)] * 4,
        out_specs=pl.BlockSpec(memory_space=pltpu.VMEM),
        scratch_shapes=[
            pltpu.VMEM((2 * N_STEP, CHUNK, D_OUT), jnp.float32),
            pltpu.SemaphoreType.DMA((2 * N_STEP,)),
            pltpu.SemaphoreType.DMA((2 * N_STEP,)),
        ],
        compiler_params=pltpu.CompilerParams(collective_id=0),
    )(x, router_W, route_idx, web)


# device time: 115474 ns/iter; 1.9644x vs baseline; 1.8989x over previous
import jax
import jax.numpy as jnp
from jax import lax
from jax.experimental import pallas as pl
from jax.experimental.pallas import tpu as pltpu

N_DEV = 8
N_TOK = 2048
D_IN = 512
D_OUT = 1024
N_EXP = 64
E_LOCAL = 8
CHUNK = N_TOK // N_DEV
HALF = D_OUT // 2
N_STEP = N_DEV - 1


def _moe_body(x_ref, rw_ref, idx_ref, we_ref, out_ref,
              w_scr, rsR, rsL, agR, agL, stR, stL,
              rsR_s, rsR_r, rsL_s, rsL_r, agR_s, agR_r, agL_s, agL_r):
    my = lax.axis_index("i")
    left = jnp.remainder(my - 1, N_DEV)
    right = jnp.remainder(my + 1, N_DEV)

    barrier_sem = pltpu.get_barrier_semaphore()
    for nbr in (left, right):
        pl.semaphore_signal(
            barrier_sem, inc=1, device_id=(nbr,),
            device_id_type=pl.DeviceIdType.MESH,
        )

    x = x_ref[:, :]
    scores = jnp.dot(x, rw_ref[:, :], preferred_element_type=jnp.float32)
    e = jnp.exp(scores - jnp.max(scores, axis=1, keepdims=True))
    idx0 = idx_ref[:, 0:1]
    idx1 = idx_ref[:, 1:2]
    iota64 = lax.broadcasted_iota(jnp.int32, (N_TOK, N_EXP), 1)
    g0 = jnp.sum(jnp.where(iota64 == idx0, e, 0.0), axis=1, keepdims=True)
    g1 = jnp.sum(jnp.where(iota64 == idx1, e, 0.0), axis=1, keepdims=True)
    gs = g0 + g1
    iota8 = lax.broadcasted_iota(jnp.int32, (N_TOK, E_LOCAL), 1) + my * E_LOCAL
    w_scr[:, :] = (jnp.where(iota8 == idx0, g0 / gs, 0.0)
                   + jnp.where(iota8 == idx1, g1 / gs, 0.0))

    def compute_chunk(cidx):
        rows = pl.ds(cidx * CHUNK, CHUNK)
        xc = x_ref[rows, :]
        wc = w_scr[rows, :]
        acc = jnp.zeros((CHUNK, D_OUT), jnp.float32)
        for ex in range(E_LOCAL):
            xw = (xc * wc[:, ex][:, None]).astype(jnp.bfloat16)
            acc += jnp.dot(xw, we_ref[ex], preferred_element_type=jnp.float32)
        out_ref[rows, :] = acc

    def rows_of(cidx):
        return pl.ds(cidx * CHUNK, CHUNK)

    compute_chunk(my)

    pl.semaphore_wait(barrier_sem, 2)

    rs_rdmas = []
    for s in range(N_STEP):
        sendR = jnp.remainder(my - s, N_DEV)
        sendL = jnp.remainder(my + s, N_DEV)
        recvR = jnp.remainder(my - s - 1, N_DEV)
        recvL = jnp.remainder(my + s + 1, N_DEV)
        rR = pltpu.make_async_remote_copy(
            src_ref=out_ref.at[rows_of(sendR), 0:HALF],
            dst_ref=rsR.at[s], send_sem=rsR_s.at[s], recv_sem=rsR_r.at[s],
            device_id=(right,), device_id_type=pl.DeviceIdType.MESH,
        )
        rL = pltpu.make_async_remote_copy(
            src_ref=out_ref.at[rows_of(sendL), HALF:D_OUT],
            dst_ref=rsL.at[s], send_sem=rsL_s.at[s], recv_sem=rsL_r.at[s],
            device_id=(left,), device_id_type=pl.DeviceIdType.MESH,
        )
        rR.start()
        rL.start()
        rs_rdmas += [rR, rL]
        if s < 3:
            compute_chunk(recvR)
            compute_chunk(recvL)
        elif s == 3:
            compute_chunk(recvR)
        rR.wait_recv()
        out_ref[rows_of(recvR), 0:HALF] = (
            out_ref[rows_of(recvR), 0:HALF] + rsR[s])
        rL.wait_recv()
        out_ref[rows_of(recvL), HALF:D_OUT] = (
            out_ref[rows_of(recvL), HALF:D_OUT] + rsL[s])

    for r in rs_rdmas:
        r.wait_send()

    stR[:, :] = out_ref[rows_of(jnp.remainder(my + 1, N_DEV)), 0:HALF
                        ].astype(jnp.bfloat16)
    stL[:, :] = out_ref[rows_of(jnp.remainder(my - 1, N_DEV)), HALF:D_OUT
                        ].astype(jnp.bfloat16)
    ag_rdmas = []
    for t in range(N_STEP):
        rR = pltpu.make_async_remote_copy(
            src_ref=(stR if t == 0 else agR.at[t - 1]),
            dst_ref=agR.at[t], send_sem=agR_s.at[t], recv_sem=agR_r.at[t],
            device_id=(right,), device_id_type=pl.DeviceIdType.MESH,
        )
        rL = pltpu.make_async_remote_copy(
            src_ref=(stL if t == 0 else agL.at[t - 1]),
            dst_ref=agL.at[t], send_sem=agL_s.at[t], recv_sem=agL_r.at[t],
            device_id=(left,), device_id_type=pl.DeviceIdType.MESH,
        )
        rR.start()
        rL.start()
        ag_rdmas += [rR, rL]
        if t > 0:
            out_ref[rows_of(jnp.remainder(my - t + 1, N_DEV)), 0:HALF] = (
                agR[t - 1].astype(jnp.float32))
            out_ref[rows_of(jnp.remainder(my + t - 1, N_DEV)), HALF:D_OUT] = (
                agL[t - 1].astype(jnp.float32))
        rR.wait_recv()
        rL.wait_recv()
    out_ref[rows_of(jnp.remainder(my - N_STEP + 1, N_DEV)), 0:HALF] = (
        agR[N_STEP - 1].astype(jnp.float32))
    out_ref[rows_of(jnp.remainder(my + N_STEP - 1, N_DEV)), HALF:D_OUT] = (
        agL[N_STEP - 1].astype(jnp.float32))
    for r in ag_rdmas:
        r.wait_send()


def kernel(x, router_W, route_idx, expert_W):
    web = expert_W.astype(jnp.bfloat16)
    return pl.pallas_call(
        _moe_body,
        out_shape=jax.ShapeDtypeStruct((N_TOK, D_OUT), jnp.float32),
        in_specs=[pl.BlockSpec(memory_space=pltpu.VMEM)] * 4,
        out_specs=pl.BlockSpec(memory_space=pltpu.VMEM),
        scratch_shapes=[
            pltpu.VMEM((N_TOK, E_LOCAL), jnp.float32),
            pltpu.VMEM((N_STEP, CHUNK, HALF), jnp.float32),
            pltpu.VMEM((N_STEP, CHUNK, HALF), jnp.float32),
            pltpu.VMEM((N_STEP, CHUNK, HALF), jnp.bfloat16),
            pltpu.VMEM((N_STEP, CHUNK, HALF), jnp.bfloat16),
            pltpu.VMEM((CHUNK, HALF), jnp.bfloat16),
            pltpu.VMEM((CHUNK, HALF), jnp.bfloat16),
            pltpu.SemaphoreType.DMA((N_STEP,)),
            pltpu.SemaphoreType.DMA((N_STEP,)),
            pltpu.SemaphoreType.DMA((N_STEP,)),
            pltpu.SemaphoreType.DMA((N_STEP,)),
            pltpu.SemaphoreType.DMA((N_STEP,)),
            pltpu.SemaphoreType.DMA((N_STEP,)),
            pltpu.SemaphoreType.DMA((N_STEP,)),
            pltpu.SemaphoreType.DMA((N_STEP,)),
        ],
        compiler_params=pltpu.CompilerParams(collective_id=0),
    )(x, router_W, route_idx, web)


# device time: 97028 ns/iter; 2.3379x vs baseline; 1.1901x over previous
import jax
import jax.numpy as jnp
from jax import lax
from jax.experimental import pallas as pl
from jax.experimental.pallas import tpu as pltpu

N_DEV = 8
N_TOK = 2048
D_IN = 512
D_OUT = 1024
N_EXP = 64
E_LOCAL = 8
CHUNK = N_TOK // N_DEV
HALF = D_OUT // 2
N_STEP = N_DEV - 1


def _moe_body(x_ref, rw_ref, idx_ref, we_ref, out_ref,
              w_scr, rsR, rsL, agR, agL, stR, stL, sgR, sgL,
              rsR_s, rsR_r, rsL_s, rsL_r, agR_s, agR_r, agL_s, agL_r):
    my = lax.axis_index("i")
    left = jnp.remainder(my - 1, N_DEV)
    right = jnp.remainder(my + 1, N_DEV)

    barrier_sem = pltpu.get_barrier_semaphore()
    for nbr in (left, right):
        pl.semaphore_signal(
            barrier_sem, inc=1, device_id=(nbr,),
            device_id_type=pl.DeviceIdType.MESH,
        )

    x = x_ref[:, :]
    scores = jnp.dot(x, rw_ref[:, :], preferred_element_type=jnp.float32)
    e = jnp.exp(scores - jnp.max(scores, axis=1, keepdims=True))
    idx0 = idx_ref[:, 0:1]
    idx1 = idx_ref[:, 1:2]
    iota64 = lax.broadcasted_iota(jnp.int32, (N_TOK, N_EXP), 1)
    g0 = jnp.sum(jnp.where(iota64 == idx0, e, 0.0), axis=1, keepdims=True)
    g1 = jnp.sum(jnp.where(iota64 == idx1, e, 0.0), axis=1, keepdims=True)
    gs = g0 + g1
    iota8 = lax.broadcasted_iota(jnp.int32, (N_TOK, E_LOCAL), 1) + my * E_LOCAL
    w_scr[:, :] = (jnp.where(iota8 == idx0, g0 / gs, 0.0)
                   + jnp.where(iota8 == idx1, g1 / gs, 0.0))

    def compute_chunk(cidx):
        rows = pl.ds(cidx * CHUNK, CHUNK)
        xc = x_ref[rows, :]
        wc = w_scr[rows, :]
        acc = jnp.zeros((CHUNK, D_OUT), jnp.float32)
        for ex in range(E_LOCAL):
            xw = (xc * wc[:, ex][:, None]).astype(jnp.bfloat16)
            acc += jnp.dot(xw, we_ref[ex], preferred_element_type=jnp.float32)
        out_ref[rows, :] = acc

    def rows_of(cidx):
        return pl.ds(cidx * CHUNK, CHUNK)

    compute_chunk(my)

    pl.semaphore_wait(barrier_sem, 2)

    rs_rdmas = []
    for s in range(N_STEP):
        sendR = jnp.remainder(my - s, N_DEV)
        sendL = jnp.remainder(my + s, N_DEV)
        recvR = jnp.remainder(my - s - 1, N_DEV)
        recvL = jnp.remainder(my + s + 1, N_DEV)
        sgR[s] = out_ref[rows_of(sendR), 0:HALF].astype(jnp.bfloat16)
        sgL[s] = out_ref[rows_of(sendL), HALF:D_OUT].astype(jnp.bfloat16)
        rR = pltpu.make_async_remote_copy(
            src_ref=sgR.at[s],
            dst_ref=rsR.at[s], send_sem=rsR_s.at[s], recv_sem=rsR_r.at[s],
            device_id=(right,), device_id_type=pl.DeviceIdType.MESH,
        )
        rL = pltpu.make_async_remote_copy(
            src_ref=sgL.at[s],
            dst_ref=rsL.at[s], send_sem=rsL_s.at[s], recv_sem=rsL_r.at[s],
            device_id=(left,), device_id_type=pl.DeviceIdType.MESH,
        )
        rR.start()
        rL.start()
        rs_rdmas += [rR, rL]
        if s < 3:
            compute_chunk(recvR)
            compute_chunk(recvL)
        elif s == 3:
            compute_chunk(recvR)
        rR.wait_recv()
        out_ref[rows_of(recvR), 0:HALF] = (
            out_ref[rows_of(recvR), 0:HALF] + rsR[s])
        rL.wait_recv()
        out_ref[rows_of(recvL), HALF:D_OUT] = (
            out_ref[rows_of(recvL), HALF:D_OUT] + rsL[s])

    for r in rs_rdmas:
        r.wait_send()

    stR[:, :] = out_ref[rows_of(jnp.remainder(my + 1, N_DEV)), 0:HALF
                        ].astype(jnp.bfloat16)
    stL[:, :] = out_ref[rows_of(jnp.remainder(my - 1, N_DEV)), HALF:D_OUT
                        ].astype(jnp.bfloat16)
    ag_rdmas = []
    for t in range(N_STEP):
        rR = pltpu.make_async_remote_copy(
            src_ref=(stR if t == 0 else agR.at[t - 1]),
            dst_ref=agR.at[t], send_sem=agR_s.at[t], recv_sem=agR_r.at[t],
            device_id=(right,), device_id_type=pl.DeviceIdType.MESH,
        )
        rL = pltpu.make_async_remote_copy(
            src_ref=(stL if t == 0 else agL.at[t - 1]),
            dst_ref=agL.at[t], send_sem=agL_s.at[t], recv_sem=agL_r.at[t],
            device_id=(left,), device_id_type=pl.DeviceIdType.MESH,
        )
        rR.start()
        rL.start()
        ag_rdmas += [rR, rL]
        if t > 0:
            out_ref[rows_of(jnp.remainder(my - t + 1, N_DEV)), 0:HALF] = (
                agR[t - 1].astype(jnp.float32))
            out_ref[rows_of(jnp.remainder(my + t - 1, N_DEV)), HALF:D_OUT] = (
                agL[t - 1].astype(jnp.float32))
        rR.wait_recv()
        rL.wait_recv()
    out_ref[rows_of(jnp.remainder(my - N_STEP + 1, N_DEV)), 0:HALF] = (
        agR[N_STEP - 1].astype(jnp.float32))
    out_ref[rows_of(jnp.remainder(my + N_STEP - 1, N_DEV)), HALF:D_OUT] = (
        agL[N_STEP - 1].astype(jnp.float32))
    for r in ag_rdmas:
        r.wait_send()


def kernel(x, router_W, route_idx, expert_W):
    web = expert_W.astype(jnp.bfloat16)
    return pl.pallas_call(
        _moe_body,
        out_shape=jax.ShapeDtypeStruct((N_TOK, D_OUT), jnp.float32),
        in_specs=[pl.BlockSpec(memory_space=pltpu.VMEM)] * 4,
        out_specs=pl.BlockSpec(memory_space=pltpu.VMEM),
        scratch_shapes=[
            pltpu.VMEM((N_TOK, E_LOCAL), jnp.float32),
            pltpu.VMEM((N_STEP, CHUNK, HALF), jnp.bfloat16),
            pltpu.VMEM((N_STEP, CHUNK, HALF), jnp.bfloat16),
            pltpu.VMEM((N_STEP, CHUNK, HALF), jnp.bfloat16),
            pltpu.VMEM((N_STEP, CHUNK, HALF), jnp.bfloat16),
            pltpu.VMEM((CHUNK, HALF), jnp.bfloat16),
            pltpu.VMEM((CHUNK, HALF), jnp.bfloat16),
            pltpu.VMEM((N_STEP, CHUNK, HALF), jnp.bfloat16),
            pltpu.VMEM((N_STEP, CHUNK, HALF), jnp.bfloat16),
            pltpu.SemaphoreType.DMA((N_STEP,)),
            pltpu.SemaphoreType.DMA((N_STEP,)),
            pltpu.SemaphoreType.DMA((N_STEP,)),
            pltpu.SemaphoreType.DMA((N_STEP,)),
            pltpu.SemaphoreType.DMA((N_STEP,)),
            pltpu.SemaphoreType.DMA((N_STEP,)),
            pltpu.SemaphoreType.DMA((N_STEP,)),
            pltpu.SemaphoreType.DMA((N_STEP,)),
        ],
        compiler_params=pltpu.CompilerParams(collective_id=0),
    )(x, router_W, route_idx, web)


# device time: 85095 ns/iter; 2.6657x vs baseline; 1.1402x over previous
import jax
import jax.numpy as jnp
from jax import lax
from jax.experimental import pallas as pl
from jax.experimental.pallas import tpu as pltpu

N_DEV = 8
N_TOK = 2048
D_IN = 512
D_OUT = 1024
N_EXP = 64
E_LOCAL = 8
CHUNK = N_TOK // N_DEV
HALF = D_OUT // 2
N_STEP = N_DEV - 1
NSUB = 4
SUBROWS = CHUNK // NSUB


def _moe_body(x_ref, rw_ref, idx_ref, we_ref, out_ref,
              w_scr, rsR, rsL, agR, agL, stR, stL, sgR, sgL,
              rsR_s, rsR_r, rsL_s, rsL_r, agR_s, agR_r, agL_s, agL_r):
    my = lax.axis_index("i")
    left = jnp.remainder(my - 1, N_DEV)
    right = jnp.remainder(my + 1, N_DEV)

    barrier_sem = pltpu.get_barrier_semaphore()
    for nbr in (left, right):
        pl.semaphore_signal(
            barrier_sem, inc=1, device_id=(nbr,),
            device_id_type=pl.DeviceIdType.MESH,
        )

    x = x_ref[:, :]
    scores = jnp.dot(x, rw_ref[:, :], preferred_element_type=jnp.float32)
    e = jnp.exp(scores - jnp.max(scores, axis=1, keepdims=True))
    idx0 = idx_ref[:, 0:1]
    idx1 = idx_ref[:, 1:2]
    iota64 = lax.broadcasted_iota(jnp.int32, (N_TOK, N_EXP), 1)
    g0 = jnp.sum(jnp.where(iota64 == idx0, e, 0.0), axis=1, keepdims=True)
    g1 = jnp.sum(jnp.where(iota64 == idx1, e, 0.0), axis=1, keepdims=True)
    gs = g0 + g1
    iota8 = lax.broadcasted_iota(jnp.int32, (N_TOK, E_LOCAL), 1) + my * E_LOCAL
    w_scr[:, :] = (jnp.where(iota8 == idx0, g0 / gs, 0.0)
                   + jnp.where(iota8 == idx1, g1 / gs, 0.0))

    def compute_chunk(cidx):
        rows = pl.ds(cidx * CHUNK, CHUNK)
        xc = x_ref[rows, :]
        wc = w_scr[rows, :]
        acc = jnp.zeros((CHUNK, D_OUT), jnp.float32)
        for ex in range(E_LOCAL):
            xw = (xc * wc[:, ex][:, None]).astype(jnp.bfloat16)
            acc += jnp.dot(xw, we_ref[ex], preferred_element_type=jnp.float32)
        out_ref[rows, :] = acc

    def rows_of(cidx):
        return pl.ds(cidx * CHUNK, CHUNK)

    compute_chunk(my)

    pl.semaphore_wait(barrier_sem, 2)

    rs_rdmas = []
    for s in range(N_STEP):
        sendR = jnp.remainder(my - s, N_DEV)
        sendL = jnp.remainder(my + s, N_DEV)
        recvR = jnp.remainder(my - s - 1, N_DEV)
        recvL = jnp.remainder(my + s + 1, N_DEV)
        sgR[s] = out_ref[rows_of(sendR), 0:HALF].astype(jnp.bfloat16)
        sgL[s] = out_ref[rows_of(sendL), HALF:D_OUT].astype(jnp.bfloat16)
        rR = pltpu.make_async_remote_copy(
            src_ref=sgR.at[s],
            dst_ref=rsR.at[s], send_sem=rsR_s.at[s], recv_sem=rsR_r.at[s],
            device_id=(right,), device_id_type=pl.DeviceIdType.MESH,
        )
        rL = pltpu.make_async_remote_copy(
            src_ref=sgL.at[s],
            dst_ref=rsL.at[s], send_sem=rsL_s.at[s], recv_sem=rsL_r.at[s],
            device_id=(left,), device_id_type=pl.DeviceIdType.MESH,
        )
        rR.start()
        rL.start()
        rs_rdmas += [rR, rL]
        if s < 3:
            compute_chunk(recvR)
            compute_chunk(recvL)
        elif s == 3:
            compute_chunk(recvR)
        rR.wait_recv()
        out_ref[rows_of(recvR), 0:HALF] = (
            out_ref[rows_of(recvR), 0:HALF] + rsR[s])
        rL.wait_recv()
        out_ref[rows_of(recvL), HALF:D_OUT] = (
            out_ref[rows_of(recvL), HALF:D_OUT] + rsL[s])

    for r in rs_rdmas:
        r.wait_send()

    stR[:, :] = out_ref[rows_of(jnp.remainder(my + 1, N_DEV)), 0:HALF
                        ].astype(jnp.bfloat16)
    stL[:, :] = out_ref[rows_of(jnp.remainder(my - 1, N_DEV)), HALF:D_OUT
                        ].astype(jnp.bfloat16)

    def ag_rdma(t, p, src, sems_s, sems_r, dst, dev):
        sub = pl.ds(p * SUBROWS, SUBROWS)
        return pltpu.make_async_remote_copy(
            src_ref=src.at[sub] if t == 0 else src.at[t - 1, sub],
            dst_ref=dst.at[t, sub],
            send_sem=sems_s.at[t * NSUB + p], recv_sem=sems_r.at[t * NSUB + p],
            device_id=(dev,), device_id_type=pl.DeviceIdType.MESH,
        )

    ag_rdmas = []
    prevR = []
    prevL = []
    for p in range(NSUB):
        rR = ag_rdma(0, p, stR, agR_s, agR_r, agR, right)
        rL = ag_rdma(0, p, stL, agL_s, agL_r, agL, left)
        rR.start()
        rL.start()
        ag_rdmas += [rR, rL]
        prevR.append(rR)
        prevL.append(rL)
    for t in range(1, N_STEP):
        curR = []
        curL = []
        for p in range(NSUB):
            prevR[p].wait_recv()
            rR = ag_rdma(t, p, agR, agR_s, agR_r, agR, right)
            rR.start()
            prevL[p].wait_recv()
            rL = ag_rdma(t, p, agL, agL_s, agL_r, agL, left)
            rL.start()
            ag_rdmas += [rR, rL]
            curR.append(rR)
            curL.append(rL)
        out_ref[rows_of(jnp.remainder(my - t + 1, N_DEV)), 0:HALF] = (
            agR[t - 1].astype(jnp.float32))
        out_ref[rows_of(jnp.remainder(my + t - 1, N_DEV)), HALF:D_OUT] = (
            agL[t - 1].astype(jnp.float32))
        prevR = curR
        prevL = curL
    for p in range(NSUB):
        prevR[p].wait_recv()
        prevL[p].wait_recv()
    out_ref[rows_of(jnp.remainder(my - N_STEP + 1, N_DEV)), 0:HALF] = (
        agR[N_STEP - 1].astype(jnp.float32))
    out_ref[rows_of(jnp.remainder(my + N_STEP - 1, N_DEV)), HALF:D_OUT] = (
        agL[N_STEP - 1].astype(jnp.float32))
    for r in ag_rdmas:
        r.wait_send()


def kernel(x, router_W, route_idx, expert_W):
    web = expert_W.astype(jnp.bfloat16)
    return pl.pallas_call(
        _moe_body,
        out_shape=jax.ShapeDtypeStruct((N_TOK, D_OUT), jnp.float32),
        in_specs=[pl.BlockSpec(memory_space=pltpu.VMEM)] * 4,
        out_specs=pl.BlockSpec(memory_space=pltpu.VMEM),
        scratch_shapes=[
            pltpu.VMEM((N_TOK, E_LOCAL), jnp.float32),
            pltpu.VMEM((N_STEP, CHUNK, HALF), jnp.bfloat16),
            pltpu.VMEM((N_STEP, CHUNK, HALF), jnp.bfloat16),
            pltpu.VMEM((N_STEP, CHUNK, HALF), jnp.bfloat16),
            pltpu.VMEM((N_STEP, CHUNK, HALF), jnp.bfloat16),
            pltpu.VMEM((CHUNK, HALF), jnp.bfloat16),
            pltpu.VMEM((CHUNK, HALF), jnp.bfloat16),
            pltpu.VMEM((N_STEP, CHUNK, HALF), jnp.bfloat16),
            pltpu.VMEM((N_STEP, CHUNK, HALF), jnp.bfloat16),
            pltpu.SemaphoreType.DMA((N_STEP,)),
            pltpu.SemaphoreType.DMA((N_STEP,)),
            pltpu.SemaphoreType.DMA((N_STEP,)),
            pltpu.SemaphoreType.DMA((N_STEP,)),
            pltpu.SemaphoreType.DMA((N_STEP * NSUB,)),
            pltpu.SemaphoreType.DMA((N_STEP * NSUB,)),
            pltpu.SemaphoreType.DMA((N_STEP * NSUB,)),
            pltpu.SemaphoreType.DMA((N_STEP * NSUB,)),
        ],
        compiler_params=pltpu.CompilerParams(collective_id=0),
    )(x, router_W, route_idx, web)


# device time: 82138 ns/iter; 2.7617x vs baseline; 1.0360x over previous
import jax
import jax.numpy as jnp
from jax import lax
from jax.experimental import pallas as pl
from jax.experimental.pallas import tpu as pltpu

N_DEV = 8
N_TOK = 2048
D_IN = 512
D_OUT = 1024
N_EXP = 64
E_LOCAL = 8
CHUNK = N_TOK // N_DEV
HALF = D_OUT // 2
N_STEP = N_DEV - 1
NSUB = 4
SUBROWS = CHUNK // NSUB


def _moe_body(x_ref, rw_ref, idx_ref, we_ref, out_ref,
              w_scr, rsR, rsL, agR, agL, stR, stL, sgR, sgL,
              rsR_s, rsR_r, rsL_s, rsL_r, agR_s, agR_r, agL_s, agL_r):
    my = lax.axis_index("i")
    left = jnp.remainder(my - 1, N_DEV)
    right = jnp.remainder(my + 1, N_DEV)

    barrier_sem = pltpu.get_barrier_semaphore()
    for nbr in (left, right):
        pl.semaphore_signal(
            barrier_sem, inc=1, device_id=(nbr,),
            device_id_type=pl.DeviceIdType.MESH,
        )

    x = x_ref[:, :]
    scores = jnp.dot(x, rw_ref[:, :], preferred_element_type=jnp.float32)
    e = jnp.exp(scores - jnp.max(scores, axis=1, keepdims=True))
    idx0 = idx_ref[:, 0:1]
    idx1 = idx_ref[:, 1:2]
    iota64 = lax.broadcasted_iota(jnp.int32, (N_TOK, N_EXP), 1)
    g0 = jnp.sum(jnp.where(iota64 == idx0, e, 0.0), axis=1, keepdims=True)
    g1 = jnp.sum(jnp.where(iota64 == idx1, e, 0.0), axis=1, keepdims=True)
    gs = g0 + g1
    iota8 = lax.broadcasted_iota(jnp.int32, (N_TOK, E_LOCAL), 1) + my * E_LOCAL
    w_scr[:, :] = (jnp.where(iota8 == idx0, g0 / gs, 0.0)
                   + jnp.where(iota8 == idx1, g1 / gs, 0.0))

    def compute_half(cidx, lo):
        rows = pl.ds(cidx * CHUNK, CHUNK)
        xc = x_ref[rows, :]
        wc = w_scr[rows, :]
        acc = jnp.zeros((CHUNK, HALF), jnp.float32)
        for ex in range(E_LOCAL):
            xw = (xc * wc[:, ex][:, None]).astype(jnp.bfloat16)
            acc += jnp.dot(xw, we_ref[ex, :, lo:lo + HALF],
                           preferred_element_type=jnp.float32)
        out_ref[rows, lo:lo + HALF] = acc

    def rows_of(cidx):
        return pl.ds(cidx * CHUNK, CHUNK)

    compute_half(my, 0)
    compute_half(my, HALF)

    pl.semaphore_wait(barrier_sem, 2)

    rs_rdmas = []
    for s in range(N_STEP):
        sendR = jnp.remainder(my - s, N_DEV)
        sendL = jnp.remainder(my + s, N_DEV)
        recvR = jnp.remainder(my - s - 1, N_DEV)
        recvL = jnp.remainder(my + s + 1, N_DEV)
        sgR[s] = out_ref[rows_of(sendR), 0:HALF].astype(jnp.bfloat16)
        sgL[s] = out_ref[rows_of(sendL), HALF:D_OUT].astype(jnp.bfloat16)
        rR = pltpu.make_async_remote_copy(
            src_ref=sgR.at[s],
            dst_ref=rsR.at[s], send_sem=rsR_s.at[s], recv_sem=rsR_r.at[s],
            device_id=(right,), device_id_type=pl.DeviceIdType.MESH,
        )
        rL = pltpu.make_async_remote_copy(
            src_ref=sgL.at[s],
            dst_ref=rsL.at[s], send_sem=rsL_s.at[s], recv_sem=rsL_r.at[s],
            device_id=(left,), device_id_type=pl.DeviceIdType.MESH,
        )
        rR.start()
        rL.start()
        rs_rdmas += [rR, rL]
        compute_half(recvR, 0)
        compute_half(recvL, HALF)
        rR.wait_recv()
        out_ref[rows_of(recvR), 0:HALF] = (
            out_ref[rows_of(recvR), 0:HALF] + rsR[s])
        rL.wait_recv()
        out_ref[rows_of(recvL), HALF:D_OUT] = (
            out_ref[rows_of(recvL), HALF:D_OUT] + rsL[s])

    for r in rs_rdmas:
        r.wait_send()

    stR[:, :] = out_ref[rows_of(jnp.remainder(my + 1, N_DEV)), 0:HALF
                        ].astype(jnp.bfloat16)
    stL[:, :] = out_ref[rows_of(jnp.remainder(my - 1, N_DEV)), HALF:D_OUT
                        ].astype(jnp.bfloat16)

    def ag_rdma(t, p, src, sems_s, sems_r, dst, dev):
        sub = pl.ds(p * SUBROWS, SUBROWS)
        return pltpu.make_async_remote_copy(
            src_ref=src.at[sub] if t == 0 else src.at[t - 1, sub],
            dst_ref=dst.at[t, sub],
            send_sem=sems_s.at[t * NSUB + p], recv_sem=sems_r.at[t * NSUB + p],
            device_id=(dev,), device_id_type=pl.DeviceIdType.MESH,
        )

    ag_rdmas = []
    prevR = []
    prevL = []
    for p in range(NSUB):
        rR = ag_rdma(0, p, stR, agR_s, agR_r, agR, right)
        rL = ag_rdma(0, p, stL, agL_s, agL_r, agL, left)
        rR.start()
        rL.start()
        ag_rdmas += [rR, rL]
        prevR.append(rR)
        prevL.append(rL)
    for t in range(1, N_STEP):
        curR = []
        curL = []
        for p in range(NSUB):
            prevR[p].wait_recv()
            rR = ag_rdma(t, p, agR, agR_s, agR_r, agR, right)
            rR.start()
            prevL[p].wait_recv()
            rL = ag_rdma(t, p, agL, agL_s, agL_r, agL, left)
            rL.start()
            ag_rdmas += [rR, rL]
            curR.append(rR)
            curL.append(rL)
        out_ref[rows_of(jnp.remainder(my - t + 1, N_DEV)), 0:HALF] = (
            agR[t - 1].astype(jnp.float32))
        out_ref[rows_of(jnp.remainder(my + t - 1, N_DEV)), HALF:D_OUT] = (
            agL[t - 1].astype(jnp.float32))
        prevR = curR
        prevL = curL
    for p in range(NSUB):
        prevR[p].wait_recv()
        prevL[p].wait_recv()
    out_ref[rows_of(jnp.remainder(my - N_STEP + 1, N_DEV)), 0:HALF] = (
        agR[N_STEP - 1].astype(jnp.float32))
    out_ref[rows_of(jnp.remainder(my + N_STEP - 1, N_DEV)), HALF:D_OUT] = (
        agL[N_STEP - 1].astype(jnp.float32))
    for r in ag_rdmas:
        r.wait_send()


def kernel(x, router_W, route_idx, expert_W):
    web = expert_W.astype(jnp.bfloat16)
    return pl.pallas_call(
        _moe_body,
        out_shape=jax.ShapeDtypeStruct((N_TOK, D_OUT), jnp.float32),
        in_specs=[pl.BlockSpec(memory_space=pltpu.VMEM)] * 4,
        out_specs=pl.BlockSpec(memory_space=pltpu.VMEM),
        scratch_shapes=[
            pltpu.VMEM((N_TOK, E_LOCAL), jnp.float32),
            pltpu.VMEM((N_STEP, CHUNK, HALF), jnp.bfloat16),
            pltpu.VMEM((N_STEP, CHUNK, HALF), jnp.bfloat16),
            pltpu.VMEM((N_STEP, CHUNK, HALF), jnp.bfloat16),
            pltpu.VMEM((N_STEP, CHUNK, HALF), jnp.bfloat16),
            pltpu.VMEM((CHUNK, HALF), jnp.bfloat16),
            pltpu.VMEM((CHUNK, HALF), jnp.bfloat16),
            pltpu.VMEM((N_STEP, CHUNK, HALF), jnp.bfloat16),
            pltpu.VMEM((N_STEP, CHUNK, HALF), jnp.bfloat16),
            pltpu.SemaphoreType.DMA((N_STEP,)),
            pltpu.SemaphoreType.DMA((N_STEP,)),
            pltpu.SemaphoreType.DMA((N_STEP,)),
            pltpu.SemaphoreType.DMA((N_STEP,)),
            pltpu.SemaphoreType.DMA((N_STEP * NSUB,)),
            pltpu.SemaphoreType.DMA((N_STEP * NSUB,)),
            pltpu.SemaphoreType.DMA((N_STEP * NSUB,)),
            pltpu.SemaphoreType.DMA((N_STEP * NSUB,)),
        ],
        compiler_params=pltpu.CompilerParams(collective_id=0),
    )(x, router_W, route_idx, web)


# device time: 82115 ns/iter; 2.7624x vs baseline; 1.0003x over previous
import jax
import jax.numpy as jnp
from jax import lax
from jax.experimental import pallas as pl
from jax.experimental.pallas import tpu as pltpu

N_DEV = 8
N_TOK = 2048
D_IN = 512
D_OUT = 1024
N_EXP = 64
E_LOCAL = 8
CHUNK = N_TOK // N_DEV
HALF = D_OUT // 2
N_STEP = N_DEV - 1
NSUB = 4
SUBROWS = CHUNK // NSUB


def _moe_body(x_ref, rw_ref, idx_ref, we_ref, out_ref,
              w_scr, xbf_scr, rsR, rsL, agR, agL, stR, stL, sgR, sgL,
              rsR_s, rsR_r, rsL_s, rsL_r, agR_s, agR_r, agL_s, agL_r):
    my = lax.axis_index("i")
    left = jnp.remainder(my - 1, N_DEV)
    right = jnp.remainder(my + 1, N_DEV)

    barrier_sem = pltpu.get_barrier_semaphore()
    for nbr in (left, right):
        pl.semaphore_signal(
            barrier_sem, inc=1, device_id=(nbr,),
            device_id_type=pl.DeviceIdType.MESH,
        )

    x = x_ref[:, :]
    scores = jnp.dot(x, rw_ref[:, :], preferred_element_type=jnp.float32)
    e = jnp.exp(scores - jnp.max(scores, axis=1, keepdims=True))
    idx0 = idx_ref[:, 0:1]
    idx1 = idx_ref[:, 1:2]
    iota64 = lax.broadcasted_iota(jnp.int32, (N_TOK, N_EXP), 1)
    g0 = jnp.sum(jnp.where(iota64 == idx0, e, 0.0), axis=1, keepdims=True)
    g1 = jnp.sum(jnp.where(iota64 == idx1, e, 0.0), axis=1, keepdims=True)
    gs = g0 + g1
    iota8 = lax.broadcasted_iota(jnp.int32, (N_TOK, E_LOCAL), 1) + my * E_LOCAL
    w_scr[:, :] = (jnp.where(iota8 == idx0, g0 / gs, 0.0)
                   + jnp.where(iota8 == idx1, g1 / gs, 0.0)
                   ).astype(jnp.bfloat16)
    xbf_scr[:, :] = x.astype(jnp.bfloat16)

    def compute_half(cidx, lo):
        rows = pl.ds(cidx * CHUNK, CHUNK)
        xc = xbf_scr[rows, :]
        wc = w_scr[rows, :]
        acc = jnp.zeros((CHUNK, HALF), jnp.float32)
        for ex in range(E_LOCAL):
            xw = xc * wc[:, ex][:, None]
            acc += jnp.dot(xw, we_ref[ex, :, lo:lo + HALF],
                           preferred_element_type=jnp.float32)
        out_ref[rows, lo:lo + HALF] = acc

    def rows_of(cidx):
        return pl.ds(cidx * CHUNK, CHUNK)

    compute_half(my, 0)
    compute_half(my, HALF)

    pl.semaphore_wait(barrier_sem, 2)

    rs_rdmas = []
    for s in range(N_STEP):
        sendR = jnp.remainder(my - s, N_DEV)
        sendL = jnp.remainder(my + s, N_DEV)
        recvR = jnp.remainder(my - s - 1, N_DEV)
        recvL = jnp.remainder(my + s + 1, N_DEV)
        sgR[s] = out_ref[rows_of(sendR), 0:HALF].astype(jnp.bfloat16)
        sgL[s] = out_ref[rows_of(sendL), HALF:D_OUT].astype(jnp.bfloat16)
        rR = pltpu.make_async_remote_copy(
            src_ref=sgR.at[s],
            dst_ref=rsR.at[s], send_sem=rsR_s.at[s], recv_sem=rsR_r.at[s],
            device_id=(right,), device_id_type=pl.DeviceIdType.MESH,
        )
        rL = pltpu.make_async_remote_copy(
            src_ref=sgL.at[s],
            dst_ref=rsL.at[s], send_sem=rsL_s.at[s], recv_sem=rsL_r.at[s],
            device_id=(left,), device_id_type=pl.DeviceIdType.MESH,
        )
        rR.start()
        rL.start()
        rs_rdmas += [rR, rL]
        compute_half(recvR, 0)
        compute_half(recvL, HALF)
        rR.wait_recv()
        out_ref[rows_of(recvR), 0:HALF] = (
            out_ref[rows_of(recvR), 0:HALF] + rsR[s])
        rL.wait_recv()
        out_ref[rows_of(recvL), HALF:D_OUT] = (
            out_ref[rows_of(recvL), HALF:D_OUT] + rsL[s])

    for r in rs_rdmas:
        r.wait_send()

    stR[:, :] = out_ref[rows_of(jnp.remainder(my + 1, N_DEV)), 0:HALF
                        ].astype(jnp.bfloat16)
    stL[:, :] = out_ref[rows_of(jnp.remainder(my - 1, N_DEV)), HALF:D_OUT
                        ].astype(jnp.bfloat16)

    def ag_rdma(t, p, src, sems_s, sems_r, dst, dev):
        sub = pl.ds(p * SUBROWS, SUBROWS)
        return pltpu.make_async_remote_copy(
            src_ref=src.at[sub] if t == 0 else src.at[t - 1, sub],
            dst_ref=dst.at[t, sub],
            send_sem=sems_s.at[t * NSUB + p], recv_sem=sems_r.at[t * NSUB + p],
            device_id=(dev,), device_id_type=pl.DeviceIdType.MESH,
        )

    ag_rdmas = []
    prevR = []
    prevL = []
    for p in range(NSUB):
        rR = ag_rdma(0, p, stR, agR_s, agR_r, agR, right)
        rL = ag_rdma(0, p, stL, agL_s, agL_r, agL, left)
        rR.start()
        rL.start()
        ag_rdmas += [rR, rL]
        prevR.append(rR)
        prevL.append(rL)
    for t in range(1, N_STEP):
        curR = []
        curL = []
        for p in range(NSUB):
            prevR[p].wait_recv()
            rR = ag_rdma(t, p, agR, agR_s, agR_r, agR, right)
            rR.start()
            prevL[p].wait_recv()
            rL = ag_rdma(t, p, agL, agL_s, agL_r, agL, left)
            rL.start()
            ag_rdmas += [rR, rL]
            curR.append(rR)
            curL.append(rL)
        out_ref[rows_of(jnp.remainder(my - t + 1, N_DEV)), 0:HALF] = (
            agR[t - 1].astype(jnp.float32))
        out_ref[rows_of(jnp.remainder(my + t - 1, N_DEV)), HALF:D_OUT] = (
            agL[t - 1].astype(jnp.float32))
        prevR = curR
        prevL = curL
    for p in range(NSUB):
        prevR[p].wait_recv()
        prevL[p].wait_recv()
    out_ref[rows_of(jnp.remainder(my - N_STEP + 1, N_DEV)), 0:HALF] = (
        agR[N_STEP - 1].astype(jnp.float32))
    out_ref[rows_of(jnp.remainder(my + N_STEP - 1, N_DEV)), HALF:D_OUT] = (
        agL[N_STEP - 1].astype(jnp.float32))
    for r in ag_rdmas:
        r.wait_send()


def kernel(x, router_W, route_idx, expert_W):
    web = expert_W.astype(jnp.bfloat16)
    return pl.pallas_call(
        _moe_body,
        out_shape=jax.ShapeDtypeStruct((N_TOK, D_OUT), jnp.float32),
        in_specs=[pl.BlockSpec(memory_space=pltpu.VMEM)] * 4,
        out_specs=pl.BlockSpec(memory_space=pltpu.VMEM),
        scratch_shapes=[
            pltpu.VMEM((N_TOK, E_LOCAL), jnp.bfloat16),
            pltpu.VMEM((N_TOK, D_IN), jnp.bfloat16),
            pltpu.VMEM((N_STEP, CHUNK, HALF), jnp.bfloat16),
            pltpu.VMEM((N_STEP, CHUNK, HALF), jnp.bfloat16),
            pltpu.VMEM((N_STEP, CHUNK, HALF), jnp.bfloat16),
            pltpu.VMEM((N_STEP, CHUNK, HALF), jnp.bfloat16),
            pltpu.VMEM((CHUNK, HALF), jnp.bfloat16),
            pltpu.VMEM((CHUNK, HALF), jnp.bfloat16),
            pltpu.VMEM((N_STEP, CHUNK, HALF), jnp.bfloat16),
            pltpu.VMEM((N_STEP, CHUNK, HALF), jnp.bfloat16),
            pltpu.SemaphoreType.DMA((N_STEP,)),
            pltpu.SemaphoreType.DMA((N_STEP,)),
            pltpu.SemaphoreType.DMA((N_STEP,)),
            pltpu.SemaphoreType.DMA((N_STEP,)),
            pltpu.SemaphoreType.DMA((N_STEP * NSUB,)),
            pltpu.SemaphoreType.DMA((N_STEP * NSUB,)),
            pltpu.SemaphoreType.DMA((N_STEP * NSUB,)),
            pltpu.SemaphoreType.DMA((N_STEP * NSUB,)),
        ],
        compiler_params=pltpu.CompilerParams(collective_id=0),
    )(x, router_W, route_idx, web)


# device time: 82067 ns/iter; 2.7641x vs baseline; 1.0006x over previous
import jax
import jax.numpy as jnp
from jax import lax
from jax.experimental import pallas as pl
from jax.experimental.pallas import tpu as pltpu

N_DEV = 8
N_TOK = 2048
D_IN = 512
D_OUT = 1024
N_EXP = 64
E_LOCAL = 8
CHUNK = N_TOK // N_DEV
HALF = D_OUT // 2
N_STEP = N_DEV - 1
NSUB = 4
SUBROWS = CHUNK // NSUB


def _moe_body(x_ref, rw_ref, idx_ref, we_ref, out_ref,
              w_scr, xbf_scr, xw_scr, rsR, rsL, agR, agL, stR, stL, sgR, sgL,
              rsR_s, rsR_r, rsL_s, rsL_r, agR_s, agR_r, agL_s, agL_r):
    my = lax.axis_index("i")
    left = jnp.remainder(my - 1, N_DEV)
    right = jnp.remainder(my + 1, N_DEV)

    barrier_sem = pltpu.get_barrier_semaphore()
    for nbr in (left, right):
        pl.semaphore_signal(
            barrier_sem, inc=1, device_id=(nbr,),
            device_id_type=pl.DeviceIdType.MESH,
        )

    x = x_ref[:, :]
    scores = jnp.dot(x, rw_ref[:, :], preferred_element_type=jnp.float32)
    e = jnp.exp(scores - jnp.max(scores, axis=1, keepdims=True))
    idx0 = idx_ref[:, 0:1]
    idx1 = idx_ref[:, 1:2]
    iota64 = lax.broadcasted_iota(jnp.int32, (N_TOK, N_EXP), 1)
    g0 = jnp.sum(jnp.where(iota64 == idx0, e, 0.0), axis=1, keepdims=True)
    g1 = jnp.sum(jnp.where(iota64 == idx1, e, 0.0), axis=1, keepdims=True)
    gs = g0 + g1
    iota8 = lax.broadcasted_iota(jnp.int32, (N_TOK, E_LOCAL), 1) + my * E_LOCAL
    w_scr[:, :] = (jnp.where(iota8 == idx0, g0 / gs, 0.0)
                   + jnp.where(iota8 == idx1, g1 / gs, 0.0)
                   ).astype(jnp.bfloat16)
    xbf_scr[:, :] = x.astype(jnp.bfloat16)

    def compute_half(cidx, lo):
        rows = pl.ds(cidx * CHUNK, CHUNK)
        xc = xbf_scr[rows, :]
        wc = w_scr[rows, :]
        for ex in range(E_LOCAL):
            xw_scr[:, ex * D_IN:(ex + 1) * D_IN] = xc * wc[:, ex][:, None]
        out_ref[rows, lo:lo + HALF] = jnp.dot(
            xw_scr[:, :], we_ref[:, lo:lo + HALF],
            preferred_element_type=jnp.float32)

    def rows_of(cidx):
        return pl.ds(cidx * CHUNK, CHUNK)

    compute_half(my, 0)
    compute_half(my, HALF)

    pl.semaphore_wait(barrier_sem, 2)

    rs_rdmas = []
    for s in range(N_STEP):
        sendR = jnp.remainder(my - s, N_DEV)
        sendL = jnp.remainder(my + s, N_DEV)
        recvR = jnp.remainder(my - s - 1, N_DEV)
        recvL = jnp.remainder(my + s + 1, N_DEV)
        sgR[s] = out_ref[rows_of(sendR), 0:HALF].astype(jnp.bfloat16)
        sgL[s] = out_ref[rows_of(sendL), HALF:D_OUT].astype(jnp.bfloat16)
        rR = pltpu.make_async_remote_copy(
            src_ref=sgR.at[s],
            dst_ref=rsR.at[s], send_sem=rsR_s.at[s], recv_sem=rsR_r.at[s],
            device_id=(right,), device_id_type=pl.DeviceIdType.MESH,
        )
        rL = pltpu.make_async_remote_copy(
            src_ref=sgL.at[s],
            dst_ref=rsL.at[s], send_sem=rsL_s.at[s], recv_sem=rsL_r.at[s],
            device_id=(left,), device_id_type=pl.DeviceIdType.MESH,
        )
        rR.start()
        rL.start()
        rs_rdmas += [rR, rL]
        compute_half(recvR, 0)
        compute_half(recvL, HALF)
        rR.wait_recv()
        out_ref[rows_of(recvR), 0:HALF] = (
            out_ref[rows_of(recvR), 0:HALF] + rsR[s])
        rL.wait_recv()
        out_ref[rows_of(recvL), HALF:D_OUT] = (
            out_ref[rows_of(recvL), HALF:D_OUT] + rsL[s])

    for r in rs_rdmas:
        r.wait_send()

    stR[:, :] = out_ref[rows_of(jnp.remainder(my + 1, N_DEV)), 0:HALF
                        ].astype(jnp.bfloat16)
    stL[:, :] = out_ref[rows_of(jnp.remainder(my - 1, N_DEV)), HALF:D_OUT
                        ].astype(jnp.bfloat16)

    def ag_rdma(t, p, src, sems_s, sems_r, dst, dev):
        sub = pl.ds(p * SUBROWS, SUBROWS)
        return pltpu.make_async_remote_copy(
            src_ref=src.at[sub] if t == 0 else src.at[t - 1, sub],
            dst_ref=dst.at[t, sub],
            send_sem=sems_s.at[t * NSUB + p], recv_sem=sems_r.at[t * NSUB + p],
            device_id=(dev,), device_id_type=pl.DeviceIdType.MESH,
        )

    ag_rdmas = []
    prevR = []
    prevL = []
    for p in range(NSUB):
        rR = ag_rdma(0, p, stR, agR_s, agR_r, agR, right)
        rL = ag_rdma(0, p, stL, agL_s, agL_r, agL, left)
        rR.start()
        rL.start()
        ag_rdmas += [rR, rL]
        prevR.append(rR)
        prevL.append(rL)
    for t in range(1, N_STEP):
        curR = []
        curL = []
        for p in range(NSUB):
            prevR[p].wait_recv()
            rR = ag_rdma(t, p, agR, agR_s, agR_r, agR, right)
            rR.start()
            prevL[p].wait_recv()
            rL = ag_rdma(t, p, agL, agL_s, agL_r, agL, left)
            rL.start()
            ag_rdmas += [rR, rL]
            curR.append(rR)
            curL.append(rL)
        out_ref[rows_of(jnp.remainder(my - t + 1, N_DEV)), 0:HALF] = (
            agR[t - 1].astype(jnp.float32))
        out_ref[rows_of(jnp.remainder(my + t - 1, N_DEV)), HALF:D_OUT] = (
            agL[t - 1].astype(jnp.float32))
        prevR = curR
        prevL = curL
    for p in range(NSUB):
        prevR[p].wait_recv()
        prevL[p].wait_recv()
    out_ref[rows_of(jnp.remainder(my - N_STEP + 1, N_DEV)), 0:HALF] = (
        agR[N_STEP - 1].astype(jnp.float32))
    out_ref[rows_of(jnp.remainder(my + N_STEP - 1, N_DEV)), HALF:D_OUT] = (
        agL[N_STEP - 1].astype(jnp.float32))
    for r in ag_rdmas:
        r.wait_send()


def kernel(x, router_W, route_idx, expert_W):
    web = expert_W.astype(jnp.bfloat16).reshape(E_LOCAL * D_IN, D_OUT)
    return pl.pallas_call(
        _moe_body,
        out_shape=jax.ShapeDtypeStruct((N_TOK, D_OUT), jnp.float32),
        in_specs=[pl.BlockSpec(memory_space=pltpu.VMEM)] * 4,
        out_specs=pl.BlockSpec(memory_space=pltpu.VMEM),
        scratch_shapes=[
            pltpu.VMEM((N_TOK, E_LOCAL), jnp.bfloat16),
            pltpu.VMEM((N_TOK, D_IN), jnp.bfloat16),
            pltpu.VMEM((CHUNK, E_LOCAL * D_IN), jnp.bfloat16),
            pltpu.VMEM((N_STEP, CHUNK, HALF), jnp.bfloat16),
            pltpu.VMEM((N_STEP, CHUNK, HALF), jnp.bfloat16),
            pltpu.VMEM((N_STEP, CHUNK, HALF), jnp.bfloat16),
            pltpu.VMEM((N_STEP, CHUNK, HALF), jnp.bfloat16),
            pltpu.VMEM((CHUNK, HALF), jnp.bfloat16),
            pltpu.VMEM((CHUNK, HALF), jnp.bfloat16),
            pltpu.VMEM((N_STEP, CHUNK, HALF), jnp.bfloat16),
            pltpu.VMEM((N_STEP, CHUNK, HALF), jnp.bfloat16),
            pltpu.SemaphoreType.DMA((N_STEP,)),
            pltpu.SemaphoreType.DMA((N_STEP,)),
            pltpu.SemaphoreType.DMA((N_STEP,)),
            pltpu.SemaphoreType.DMA((N_STEP,)),
            pltpu.SemaphoreType.DMA((N_STEP * NSUB,)),
            pltpu.SemaphoreType.DMA((N_STEP * NSUB,)),
            pltpu.SemaphoreType.DMA((N_STEP * NSUB,)),
            pltpu.SemaphoreType.DMA((N_STEP * NSUB,)),
        ],
        compiler_params=pltpu.CompilerParams(collective_id=0),
    )(x, router_W, route_idx, web)
